# Initial kernel scaffold; baseline (speedup 1.0000x reference)
import jax
import jax.numpy as jnp
from jax.experimental import pallas as pl


def kernel(x, adj, W0m, b0m, W0v, b0v, W1m, b1m, W1v, b1v):
    raise NotImplementedError("write your pallas kernel here")


# speedup vs baseline: 26.9225x; 26.9225x over previous
import functools

import jax
import jax.numpy as jnp
from jax import lax
from jax.experimental import pallas as pl
from jax.experimental.pallas import tpu as pltpu
from jax.experimental.pallas import tpu_sc as plsc

_N = 10000
_D = 128
_H = 128
_C = 64
_P = 128

_NP = 10240
_NC = 2
_NS = 16
_NW = _NC * _NS
_CHUNK = 128
_ROWS_PT = _NP // _NS


def _elu(x):
    return jnp.where(x > 0, x, jnp.exp(x) - 1.0)


def _deg_body(row_hbm, out_hbm, idx_v, ones_v, zrow_v, deg_sh):
    c = lax.axis_index("c")
    s = lax.axis_index("s")
    wid = c * _NS + s
    epw = row_hbm.shape[0] // _NW
    base = wid * epw

    def _z16(i, carry):
        zrow_v[pl.ds(i * 16, 16)] = jnp.zeros((16,), jnp.float32)
        return carry

    lax.fori_loop(0, _ROWS_PT // 16, _z16, 0)
    for q in range(_CHUNK // 16):
        ones_v[pl.ds(q * 16, 16)] = jnp.ones((16,), jnp.float32)
    pltpu.sync_copy(zrow_v, deg_sh.at[pl.ds(s * _ROWS_PT, _ROWS_PT)])
    plsc.subcore_barrier()

    def _chunk(k, carry):
        pltpu.sync_copy(row_hbm.at[pl.ds(base + k * _CHUNK, _CHUNK)], idx_v)
        pltpu.sync_copy(ones_v, deg_sh.at[idx_v], add=True)
        return carry

    lax.fori_loop(0, epw // _CHUNK, _chunk, 0)
    plsc.subcore_barrier()

    pltpu.sync_copy(deg_sh.at[pl.ds(s * _ROWS_PT, _ROWS_PT)], zrow_v)
    pltpu.sync_copy(zrow_v, out_hbm.at[c, pl.ds(s * _ROWS_PT, _ROWS_PT)])


def _spmm_body(row_hbm, col_hbm, mvs_hbm, pmv_hbm,
               ridx_v, cidx_v, rows_v, zbuf_v, acc_sh, gsem):
    c = lax.axis_index("c")
    s = lax.axis_index("s")
    wid = c * _NS + s
    epw = row_hbm.shape[0] // _NW
    base = wid * epw

    def _z2d(i, carry):
        for q in range(_P // 16):
            zbuf_v[i, pl.ds(q * 16, 16)] = jnp.zeros((16,), jnp.float32)
        return carry

    lax.fori_loop(0, _CHUNK, _z2d, 0)
    for j in range(_ROWS_PT // _CHUNK):
        off = s * _ROWS_PT + j * _CHUNK
        pltpu.sync_copy(zbuf_v, acc_sh.at[pl.ds(off, _CHUNK)])
    plsc.subcore_barrier()

    def _chunk(k, carry):
        pltpu.sync_copy(row_hbm.at[pl.ds(base + k * _CHUNK, _CHUNK)], ridx_v)
        pltpu.sync_copy(col_hbm.at[pl.ds(base + k * _CHUNK, _CHUNK)], cidx_v)
        pltpu.async_copy(mvs_hbm.at[cidx_v], rows_v, gsem).wait()
        pltpu.sync_copy(rows_v, acc_sh.at[ridx_v], add=True)
        return carry

    lax.fori_loop(0, epw // _CHUNK, _chunk, 0)
    plsc.subcore_barrier()

    for j in range(_ROWS_PT // _CHUNK):
        off = s * _ROWS_PT + j * _CHUNK
        pltpu.sync_copy(acc_sh.at[pl.ds(off, _CHUNK)], zbuf_v)
        pltpu.sync_copy(zbuf_v, pmv_hbm.at[c, pl.ds(off, _CHUNK)])


def _dense_body(x_ref, deg_ref, w0m_ref, b0m_ref, w0v_ref, b0v_ref,
                w1m_ref, b1m_ref, w1v_ref, b1v_ref, mvs_ref):
    dot = functools.partial(jnp.dot, preferred_element_type=jnp.float32,
                            precision=lax.Precision.HIGHEST)
    xb = x_ref[...]
    hm = _elu(dot(xb, w0m_ref[...]) + b0m_ref[...])
    hv = jnp.maximum(dot(xb, w0v_ref[...]) + b0v_ref[...], 0.0)
    m = _elu(dot(hm, w1m_ref[...]) + b1m_ref[...])
    v = jnp.maximum(dot(hv, w1v_ref[...]) + b1v_ref[...], 0.0) + 1e-6
    att = jnp.exp(-v)
    deg = deg_ref[...]
    dinv0 = lax.rsqrt(deg)
    dinv1 = 1.0 / deg
    mvs_ref[...] = jnp.concatenate(
        [dinv0 * (m * att), dinv1 * (v * att * att)], axis=1)


def _final_body(pmv_ref, mvs_ref, deg_ref, smp_ref, out_ref):
    deg = deg_ref[...]
    dinv0 = lax.rsqrt(deg)
    dinv1 = 1.0 / deg
    tot = pmv_ref[0] + pmv_ref[1] + mvs_ref[...]
    mean = dinv0 * tot[:, :_C]
    var = dinv1 * tot[:, _C:]
    o = mean + smp_ref[...] * jnp.sqrt(var)
    o = o - jnp.max(o, axis=-1, keepdims=True)
    out_ref[...] = o - jnp.log(jnp.sum(jnp.exp(o), axis=-1, keepdims=True))


def kernel(x, adj, W0m, b0m, W0v, b0v, W1m, b1m, W1v, b1v):
    sample = jax.random.normal(jax.random.key(42), (_N, _C), dtype=jnp.float32)
    e = adj.shape[1]
    epw = -(-e // (_NW * _CHUNK)) * _CHUNK
    ep = epw * _NW

    rowp = jnp.concatenate([adj[0], jnp.full((ep - e,), _NP - 1, jnp.int32)])
    colp = jnp.concatenate([adj[1], jnp.zeros((ep - e,), jnp.int32)])

    mesh = plsc.VectorSubcoreMesh(core_axis_name="c", subcore_axis_name="s")

    deg_call = pl.kernel(
        _deg_body,
        out_type=jax.ShapeDtypeStruct((_NC, _NP), jnp.float32),
        mesh=mesh,
        scratch_types=[
            pltpu.VMEM((_CHUNK,), jnp.int32),
            pltpu.VMEM((_CHUNK,), jnp.float32),
            pltpu.VMEM((_ROWS_PT,), jnp.float32),
            pltpu.VMEM_SHARED((_NP,), jnp.float32),
        ],
    )
    pdeg = deg_call(rowp)
    deg2 = (pdeg[0, :_N] + pdeg[1, :_N] + 1.0).reshape(_N, 1)

    nb = 10
    bn = _N // nb
    mvs = pl.pallas_call(
        _dense_body,
        grid=(nb,),
        in_specs=[
            pl.BlockSpec((bn, _D), lambda i: (i, 0)),
            pl.BlockSpec((bn, 1), lambda i: (i, 0)),
            pl.BlockSpec((_D, _H), lambda i: (0, 0)),
            pl.BlockSpec((1, _H), lambda i: (0, 0)),
            pl.BlockSpec((_D, _H), lambda i: (0, 0)),
            pl.BlockSpec((1, _H), lambda i: (0, 0)),
            pl.BlockSpec((_H, _C), lambda i: (0, 0)),
            pl.BlockSpec((1, _C), lambda i: (0, 0)),
            pl.BlockSpec((_H, _C), lambda i: (0, 0)),
            pl.BlockSpec((1, _C), lambda i: (0, 0)),
        ],
        out_specs=pl.BlockSpec((bn, _P), lambda i: (i, 0)),
        out_shape=jax.ShapeDtypeStruct((_N, _P), jnp.float32),
    )(x, deg2, W0m, b0m.reshape(1, _H), W0v, b0v.reshape(1, _H),
      W1m, b1m.reshape(1, _C), W1v, b1v.reshape(1, _C))

    spmm_call = pl.kernel(
        _spmm_body,
        out_type=jax.ShapeDtypeStruct((_NC, _NP, _P), jnp.float32),
        mesh=mesh,
        scratch_types=[
            pltpu.VMEM((_CHUNK,), jnp.int32),
            pltpu.VMEM((_CHUNK,), jnp.int32),
            pltpu.VMEM((_CHUNK, _P), jnp.float32),
            pltpu.VMEM((_CHUNK, _P), jnp.float32),
            pltpu.VMEM_SHARED((_NP, _P), jnp.float32),
            pltpu.SemaphoreType.DMA,
        ],
    )
    pmv = spmm_call(rowp, colp, mvs)

    out = pl.pallas_call(
        _final_body,
        grid=(nb,),
        in_specs=[
            pl.BlockSpec((_NC, bn, _P), lambda i: (0, i, 0)),
            pl.BlockSpec((bn, _P), lambda i: (i, 0)),
            pl.BlockSpec((bn, 1), lambda i: (i, 0)),
            pl.BlockSpec((bn, _C), lambda i: (i, 0)),
        ],
        out_specs=pl.BlockSpec((bn, _C), lambda i: (i, 0)),
        out_shape=jax.ShapeDtypeStruct((_N, _C), jnp.float32),
    )(pmv, mvs, deg2, sample)
    return out
